# Initial kernel scaffold; baseline (speedup 1.0000x reference)
import jax
import jax.numpy as jnp
from jax.experimental import pallas as pl


def kernel(x, w_router, w1, w2):
    raise NotImplementedError("write your pallas kernel here")


# speedup vs baseline: 1.3899x; 1.3899x over previous
import functools

import jax
import jax.numpy as jnp
from jax import lax
from jax.experimental import pallas as pl
from jax.experimental.pallas import tpu as pltpu
from jax.experimental.pallas import tpu_sc as plsc

NE = 8
DM = 1024
DF = 2048
T = 2048
BT = 128
NT = (T * 2 + NE * (BT - 1) + BT - 1) // BT
P = NT * BT
NC, NS = 2, 16
NW = NC * NS
TPW = T // NW
KB = 256


def _route_body(x_ref, wr_ref, dsta_ref, dstb_ref, wa_ref, wb_ref, te_ref):
    x = x_ref[...]
    logits = jnp.dot(x, wr_ref[...], preferred_element_type=jnp.float32)
    m = jnp.max(logits, axis=1, keepdims=True)
    ex = jnp.exp(logits - m)
    probs = ex / jnp.sum(ex, axis=1, keepdims=True)

    eio = lax.broadcasted_iota(jnp.int32, (T, NE), 1)
    m1 = jnp.max(probs, axis=1, keepdims=True)
    i1 = jnp.min(jnp.where(probs == m1, eio, NE), axis=1, keepdims=True)
    p2 = jnp.where(eio == i1, -1.0, probs)
    m2 = jnp.max(p2, axis=1, keepdims=True)
    i2 = jnp.min(jnp.where(p2 == m2, eio, NE), axis=1, keepdims=True)
    s = m1 + m2
    wa_ref[...] = m1 / s
    wb_ref[...] = m2 / s

    oha = (eio == i1).astype(jnp.float32)
    ohb = (eio == i2).astype(jnp.float32)
    ind = oha + ohb

    pos = jnp.zeros((T, NE), jnp.float32)
    rio = lax.broadcasted_iota(jnp.int32, (T, KB), 0)
    cio = lax.broadcasted_iota(jnp.int32, (T, KB), 1)
    for kb in range(T // KB):
        tri = (rio > cio + kb * KB).astype(jnp.float32)
        pos = pos + jnp.dot(tri, ind[kb * KB:(kb + 1) * KB, :],
                            preferred_element_type=jnp.float32)

    counts = jnp.sum(ind, axis=0, keepdims=True)
    tiles = jnp.floor((counts + (BT - 1)) * (1.0 / BT))
    ii = lax.broadcasted_iota(jnp.int32, (NE, NE), 0)
    jj = lax.broadcasted_iota(jnp.int32, (NE, NE), 1)
    excl = (ii < jj).astype(jnp.float32)
    start_tiles = jnp.dot(tiles, excl, preferred_element_type=jnp.float32)

    dest = start_tiles * BT + pos
    dsta_ref[...] = jnp.sum(dest * oha, axis=1).astype(jnp.int32)
    dstb_ref[...] = jnp.sum(dest * ohb, axis=1).astype(jnp.int32)

    tio = lax.broadcasted_iota(jnp.int32, (NT, NE), 0)
    st_i = start_tiles.astype(jnp.int32)
    te_ref[...] = jnp.sum((tio >= st_i).astype(jnp.int32), axis=1) - 1


_route = pl.pallas_call(
    _route_body,
    out_shape=[
        jax.ShapeDtypeStruct((T,), jnp.int32),
        jax.ShapeDtypeStruct((T,), jnp.int32),
        jax.ShapeDtypeStruct((T, 1), jnp.float32),
        jax.ShapeDtypeStruct((T, 1), jnp.float32),
        jax.ShapeDtypeStruct((NT,), jnp.int32),
    ],
)


def _dispatch_body(x_hbm, dsta_hbm, dstb_hbm, xs_hbm, idx_v, rows_v, sem):
    wid = lax.axis_index("s") * NC + lax.axis_index("c")
    base = wid * TPW
    pltpu.sync_copy(x_hbm.at[pl.ds(base, TPW)], rows_v)
    pltpu.sync_copy(dsta_hbm.at[wid], idx_v)
    pltpu.async_copy(rows_v, xs_hbm.at[idx_v], sem).wait()
    pltpu.sync_copy(dstb_hbm.at[wid], idx_v)
    pltpu.async_copy(rows_v, xs_hbm.at[idx_v], sem).wait()


def _gmm_body(te_ref, xs_ref, w1_ref, w2_ref, out_ref):
    h = jnp.dot(xs_ref[...], w1_ref[0], preferred_element_type=jnp.float32)
    h = h * (1.0 / (1.0 + jnp.exp(-h)))
    out_ref[...] = jnp.dot(h, w2_ref[0], preferred_element_type=jnp.float32)


_gmm = pl.pallas_call(
    _gmm_body,
    grid_spec=pltpu.PrefetchScalarGridSpec(
        num_scalar_prefetch=1,
        grid=(NT,),
        in_specs=[
            pl.BlockSpec((BT, DM), lambda j, te: (j, 0)),
            pl.BlockSpec((1, DM, DF), lambda j, te: (te[j], 0, 0)),
            pl.BlockSpec((1, DF, DM), lambda j, te: (te[j], 0, 0)),
        ],
        out_specs=pl.BlockSpec((BT, DM), lambda j, te: (j, 0)),
    ),
    out_shape=jax.ShapeDtypeStruct((P, DM), jnp.float32),
)


def _gather2_body(h2_hbm, dsta_hbm, dstb_hbm, ga_hbm, gb_hbm, idx_v, buf_v, sem):
    wid = lax.axis_index("s") * NC + lax.axis_index("c")
    base = wid * TPW
    pltpu.sync_copy(dsta_hbm.at[wid], idx_v)
    pltpu.async_copy(h2_hbm.at[idx_v], buf_v, sem).wait()
    pltpu.sync_copy(buf_v, ga_hbm.at[pl.ds(base, TPW)])
    pltpu.sync_copy(dstb_hbm.at[wid], idx_v)
    pltpu.async_copy(h2_hbm.at[idx_v], buf_v, sem).wait()
    pltpu.sync_copy(buf_v, gb_hbm.at[pl.ds(base, TPW)])


def _combine_body(ga_ref, gb_ref, wa_ref, wb_ref, out_ref):
    out_ref[...] = wa_ref[...] * ga_ref[...] + wb_ref[...] * gb_ref[...]


BC = 256
_combine = pl.pallas_call(
    _combine_body,
    grid=(T // BC,),
    in_specs=[
        pl.BlockSpec((BC, DM), lambda i: (i, 0)),
        pl.BlockSpec((BC, DM), lambda i: (i, 0)),
        pl.BlockSpec((BC, 1), lambda i: (i, 0)),
        pl.BlockSpec((BC, 1), lambda i: (i, 0)),
    ],
    out_specs=pl.BlockSpec((BC, DM), lambda i: (i, 0)),
    out_shape=jax.ShapeDtypeStruct((T, DM), jnp.float32),
)


@functools.lru_cache(maxsize=1)
def _sc_kernels():
    mesh = plsc.VectorSubcoreMesh(
        core_axis_name="c", subcore_axis_name="s",
        num_cores=NC, num_subcores=NS)
    dispatch = pl.kernel(
        _dispatch_body,
        mesh=mesh,
        out_type=jax.ShapeDtypeStruct((P, DM), jnp.float32),
        scratch_types=[
            pltpu.VMEM((TPW,), jnp.int32),
            pltpu.VMEM((TPW, DM), jnp.float32),
            pltpu.SemaphoreType.DMA,
        ],
    )
    gather2 = pl.kernel(
        _gather2_body,
        mesh=mesh,
        out_type=[
            jax.ShapeDtypeStruct((T, DM), jnp.float32),
            jax.ShapeDtypeStruct((T, DM), jnp.float32),
        ],
        scratch_types=[
            pltpu.VMEM((TPW,), jnp.int32),
            pltpu.VMEM((TPW, DM), jnp.float32),
            pltpu.SemaphoreType.DMA,
        ],
    )
    return dispatch, gather2


def kernel(x, w_router, w1, w2):
    dispatch, gather2 = _sc_kernels()
    dsta, dstb, wa, wb, te = _route(x, w_router)
    da = dsta.reshape(NW, TPW)
    db = dstb.reshape(NW, TPW)
    xs = dispatch(x, da, db)
    h2 = _gmm(te, xs, w1, w2)
    ga, gb = gather2(h2, da, db)
    return _combine(ga, gb, wa, wb)
